# Initial kernel scaffold; baseline (speedup 1.0000x reference)
import jax
import jax.numpy as jnp
from jax.experimental import pallas as pl


def kernel(x, edge_index, W1, b1, g1, be1, W2, b2, g2, be2, W3, b3, g3, be3, Wg, a_s, a_d, bg, Wc1, bc1, Wc2, bc2):
    raise NotImplementedError("write your pallas kernel here")


# speedup vs baseline: 2.7678x; 2.7678x over previous
import functools

import jax
import jax.numpy as jnp
from jax import lax
from jax.experimental import pallas as pl
from jax.experimental.pallas import tpu as pltpu
from jax.experimental.pallas import tpu_sc as plsc

N = 10000
D = 128
HID = 128
HEADS = 8
C = 128

NP = 10240
DUM = 10000
EP = 360448
ERows = 2816
JB = ERows // 32
SL = NP // 16
BM = 400
BM2 = 512

_f32 = jnp.float32
_i32 = jnp.int32

_MESH = plsc.VectorSubcoreMesh(core_axis_name="c", subcore_axis_name="s")


def _wid():
    return lax.axis_index("c") * 16 + lax.axis_index("s")


@functools.partial(
    pl.kernel,
    out_type=jax.ShapeDtypeStruct((2, NP, 16), _f32),
    mesh=_MESH,
    scratch_types=[
        pltpu.VMEM((JB, 128), _i32),
        pltpu.VMEM((128, 16), _f32),
        pltpu.VMEM_SHARED((NP, 16), _f32),
    ],
)
def _sc_deg(dst2d, ones16, z16, out, dstv, ones_v, acc):
    c = lax.axis_index("c")
    s = lax.axis_index("s")
    w = _wid()
    pltpu.sync_copy(z16.at[pl.ds(s * SL, SL)], acc.at[pl.ds(s * SL, SL)])
    pltpu.sync_copy(ones16, ones_v)
    pltpu.sync_copy(dst2d.at[pl.ds(w * JB, JB)], dstv)
    plsc.subcore_barrier()

    def body(j, _):
        pltpu.sync_copy(ones_v, acc.at[dstv.at[j]], add=True)
        return ()

    lax.fori_loop(0, JB, body, (), unroll=False)
    plsc.subcore_barrier()
    pltpu.sync_copy(acc.at[pl.ds(s * SL, SL)], out.at[c, pl.ds(s * SL, SL)])


@functools.partial(
    pl.kernel,
    out_type=jax.ShapeDtypeStruct((2, NP, 128), _f32),
    mesh=_MESH,
    scratch_types=[
        pltpu.VMEM((8, 128), _i32),
        pltpu.VMEM((8, 128), _i32),
        pltpu.VMEM((128, 128), _f32),
        pltpu.VMEM_SHARED((NP, 128), _f32),
        pltpu.SemaphoreType.DMA,
    ],
)
def _sc_agg(table, src2d, dst2d, z128, out, srcv, dstv, rows, acc, sem):
    c = lax.axis_index("c")
    s = lax.axis_index("s")
    w = _wid()
    pltpu.sync_copy(z128.at[pl.ds(s * SL, SL)], acc.at[pl.ds(s * SL, SL)])
    plsc.subcore_barrier()

    def group(g, _):
        pltpu.sync_copy(src2d.at[pl.ds(w * JB + g * 8, 8)], srcv)
        pltpu.sync_copy(dst2d.at[pl.ds(w * JB + g * 8, 8)], dstv)

        def body(j, _):
            pltpu.async_copy(table.at[srcv.at[j]], rows, sem).wait()
            pltpu.sync_copy(rows, acc.at[dstv.at[j]], add=True)
            return ()

        lax.fori_loop(0, 8, body, (), unroll=False)
        return ()

    lax.fori_loop(0, JB // 8, group, (), unroll=False)
    plsc.subcore_barrier()
    pltpu.sync_copy(acc.at[pl.ds(s * SL, SL)], out.at[c, pl.ds(s * SL, SL)])


@functools.partial(
    pl.kernel,
    out_type=jax.ShapeDtypeStruct((EP, 16), _f32),
    mesh=_MESH,
    scratch_types=[
        pltpu.VMEM((JB, 128), _i32),
        pltpu.VMEM((JB, 128), _i32),
        pltpu.VMEM((128, 128), _f32),
        pltpu.VMEM((128, 128), _f32),
        pltpu.VMEM((128, 16), _f32),
        pltpu.SemaphoreType.DMA,
        pltpu.SemaphoreType.DMA,
    ],
)
def _sc_gat_ee(esX, edX, src2d, dst2d, ee_out,
               srcv, dstv, av, bv, eev, sem_a, sem_b):
    w = _wid()
    pltpu.sync_copy(src2d.at[pl.ds(w * JB, JB)], srcv)
    pltpu.sync_copy(dst2d.at[pl.ds(w * JB, JB)], dstv)

    def body(j, _):
        pltpu.async_copy(esX.at[srcv.at[j]], av, sem_a).wait()
        pltpu.async_copy(edX.at[dstv.at[j]], bv, sem_b).wait()

        def inner(i, _):
            e = av[i, pl.ds(0, 16)] + bv[i, pl.ds(0, 16)]
            e = jnp.where(e > 0.0, e, e * 0.2)
            eev[i, :] = jnp.exp(e)
            return ()

        lax.fori_loop(0, 128, inner, (), unroll=4)
        pltpu.sync_copy(eev, ee_out.at[pl.ds((w * JB + j) * 128, 128)])
        return ()

    lax.fori_loop(0, JB, body, (), unroll=False)


@functools.partial(
    pl.kernel,
    out_type=jax.ShapeDtypeStruct((2, NP, 16), _f32),
    mesh=_MESH,
    scratch_types=[
        pltpu.VMEM((JB, 128), _i32),
        pltpu.VMEM((128, 16), _f32),
        pltpu.VMEM_SHARED((NP, 16), _f32),
    ],
)
def _sc_den(ee, dst2d, z16, out, dstv, eev, acc):
    c = lax.axis_index("c")
    s = lax.axis_index("s")
    w = _wid()
    pltpu.sync_copy(z16.at[pl.ds(s * SL, SL)], acc.at[pl.ds(s * SL, SL)])
    pltpu.sync_copy(dst2d.at[pl.ds(w * JB, JB)], dstv)
    plsc.subcore_barrier()

    def body(j, _):
        pltpu.sync_copy(ee.at[pl.ds((w * JB + j) * 128, 128)], eev)
        pltpu.sync_copy(eev, acc.at[dstv.at[j]], add=True)
        return ()

    lax.fori_loop(0, JB, body, (), unroll=False)
    plsc.subcore_barrier()
    pltpu.sync_copy(acc.at[pl.ds(s * SL, SL)], out.at[c, pl.ds(s * SL, SL)])


@functools.lru_cache(maxsize=None)
def _make_gat_head(h):
    @functools.partial(
        pl.kernel,
        out_type=jax.ShapeDtypeStruct((2, NP, 128), _f32),
        mesh=_MESH,
        scratch_types=[
            pltpu.VMEM((8, 128), _i32),
            pltpu.VMEM((8, 128), _i32),
            pltpu.VMEM((128, 128), _f32),
            pltpu.VMEM((128, 16), _f32),
            pltpu.VMEM_SHARED((NP, 128), _f32),
            pltpu.SemaphoreType.DMA,
        ],
        name=f"sc_gat_head{h}",
    )
    def _sc_gat_head(hh_h, src2d, dst2d, ee, z128, out,
                     srcv, dstv, rows, eev, acc, sem):
        c = lax.axis_index("c")
        s = lax.axis_index("s")
        w = _wid()
        pltpu.sync_copy(z128.at[pl.ds(s * SL, SL)], acc.at[pl.ds(s * SL, SL)])
        plsc.subcore_barrier()

        def group(g, _):
            pltpu.sync_copy(src2d.at[pl.ds(w * JB + g * 8, 8)], srcv)
            pltpu.sync_copy(dst2d.at[pl.ds(w * JB + g * 8, 8)], dstv)

            def body(j, _):
                pltpu.async_copy(hh_h.at[srcv.at[j]], rows, sem).wait()
                pltpu.sync_copy(
                    ee.at[pl.ds((w * JB + g * 8 + j) * 128, 128)], eev)

                def inner(i, _):
                    row = eev[i, :]
                    sc = jnp.full((16,), row[h], _f32)
                    for blk in range(8):
                        rows[i, pl.ds(blk * 16, 16)] = (
                            rows[i, pl.ds(blk * 16, 16)] * sc
                        )
                    return ()

                lax.fori_loop(0, 128, inner, (), unroll=2)
                pltpu.sync_copy(rows, acc.at[dstv.at[j]], add=True)
                return ()

            lax.fori_loop(0, 8, body, (), unroll=False)
            return ()

        lax.fori_loop(0, JB // 8, group, (), unroll=False)
        plsc.subcore_barrier()
        pltpu.sync_copy(acc.at[pl.ds(s * SL, SL)],
                        out.at[c, pl.ds(s * SL, SL)])

    return _sc_gat_head


def _mm_scale_body(h_ref, w_ref, deg_ref, out_ref):
    deg = deg_ref[0, :, 0:1] + deg_ref[1, :, 0:1]
    dinv = lax.rsqrt(deg)
    out_ref[...] = jnp.dot(h_ref[...], w_ref[...],
                           preferred_element_type=_f32) * dinv


def _tc_mm_scale(h, w, degp):
    return pl.pallas_call(
        _mm_scale_body,
        grid=(N // BM,),
        in_specs=[
            pl.BlockSpec((BM, 128), lambda i: (i, 0)),
            pl.BlockSpec((128, 128), lambda i: (0, 0)),
            pl.BlockSpec((2, BM, 16), lambda i: (0, i, 0)),
        ],
        out_specs=pl.BlockSpec((BM, 128), lambda i: (i, 0)),
        out_shape=jax.ShapeDtypeStruct((N, 128), _f32),
    )(h, w, degp)


def _bnz_body(p_ref, deg_ref, b_ref, z_ref, st_ref):
    i = pl.program_id(0)
    deg = deg_ref[0, :, 0:1] + deg_ref[1, :, 0:1]
    dinv = lax.rsqrt(deg)
    z = dinv * (p_ref[0] + p_ref[1]) + b_ref[...]
    z_ref[...] = z

    @pl.when(i == 0)
    def _():
        st_ref[...] = jnp.zeros((2, 128), _f32)

    s1 = jnp.sum(z, axis=0, keepdims=True)
    s2 = jnp.sum(z * z, axis=0, keepdims=True)
    st_ref[...] += jnp.concatenate([s1, s2], axis=0)


def _tc_bnz(p, degp, b):
    return pl.pallas_call(
        _bnz_body,
        grid=(N // BM,),
        in_specs=[
            pl.BlockSpec((2, BM, 128), lambda i: (0, i, 0)),
            pl.BlockSpec((2, BM, 16), lambda i: (0, i, 0)),
            pl.BlockSpec((1, 128), lambda i: (0, 0)),
        ],
        out_specs=[
            pl.BlockSpec((BM, 128), lambda i: (i, 0)),
            pl.BlockSpec((2, 128), lambda i: (0, 0)),
        ],
        out_shape=[
            jax.ShapeDtypeStruct((N, 128), _f32),
            jax.ShapeDtypeStruct((2, 128), _f32),
        ],
    )(p, degp, b)


def _apply_body(z_ref, st_ref, g_ref, be_ref, hp_ref, out_ref):
    m = st_ref[0:1, :] * (1.0 / N)
    ex2 = st_ref[1:2, :] * (1.0 / N)
    var = ex2 - m * m
    tau = lax.rsqrt(var + 1e-5)
    hn = g_ref[...] * (z_ref[...] - m) * tau + be_ref[...]
    out_ref[...] = hp_ref[...] + jnp.maximum(hn, 0.0)


def _tc_apply(z, st, g, be, hp):
    return pl.pallas_call(
        _apply_body,
        grid=(N // BM,),
        in_specs=[
            pl.BlockSpec((BM, 128), lambda i: (i, 0)),
            pl.BlockSpec((2, 128), lambda i: (0, 0)),
            pl.BlockSpec((1, 128), lambda i: (0, 0)),
            pl.BlockSpec((1, 128), lambda i: (0, 0)),
            pl.BlockSpec((BM, 128), lambda i: (i, 0)),
        ],
        out_specs=pl.BlockSpec((BM, 128), lambda i: (i, 0)),
        out_shape=jax.ShapeDtypeStruct((N, 128), _f32),
    )(z, st, g, be, hp)


def _gat_pre_body(x_ref, wg_ref, ass_ref, asd_ref, hh_ref, es_ref, ed_ref):
    h = pl.program_id(1)
    hh = jnp.dot(x_ref[...], wg_ref[...], preferred_element_type=_f32)
    hh_ref[0] = hh

    @pl.when(h == 0)
    def _():
        es_ref[...] = jnp.zeros((BM2, 128), _f32)
        ed_ref[...] = jnp.zeros((BM2, 128), _f32)

    es_ref[...] += jnp.dot(hh, ass_ref[...], preferred_element_type=_f32)
    ed_ref[...] += jnp.dot(hh, asd_ref[...], preferred_element_type=_f32)


def _tc_gat_pre(x, wg, ass, asd):
    return pl.pallas_call(
        _gat_pre_body,
        grid=(NP // BM2, HEADS),
        in_specs=[
            pl.BlockSpec((BM2, 128), lambda i, h: (i, 0)),
            pl.BlockSpec((128, 128), lambda i, h: (0, h)),
            pl.BlockSpec((128, 128), lambda i, h: (h, 0)),
            pl.BlockSpec((128, 128), lambda i, h: (h, 0)),
        ],
        out_specs=[
            pl.BlockSpec((1, BM2, 128), lambda i, h: (h, i, 0)),
            pl.BlockSpec((BM2, 128), lambda i, h: (i, 0)),
            pl.BlockSpec((BM2, 128), lambda i, h: (i, 0)),
        ],
        out_shape=[
            jax.ShapeDtypeStruct((HEADS, NP, 128), _f32),
            jax.ShapeDtypeStruct((NP, 128), _f32),
            jax.ShapeDtypeStruct((NP, 128), _f32),
        ],
    )(x, wg, ass, asd)


def _final_body(op0, op1, op2, op3, op4, op5, op6, op7,
                den_ref, bg_ref, wc1_ref, bc1_ref, wc2_ref, bc2_ref,
                out_ref, colacc):
    i = pl.program_id(0)
    ops = (op0, op1, op2, op3, op4, op5, op6, op7)

    @pl.when(i == 0)
    def _():
        colacc[...] = jnp.zeros((1, 128), _f32)

    den = den_ref[0, :, 0:8] + den_ref[1, :, 0:8]
    rec = 1.0 / den
    s = jnp.zeros((BM, 128), _f32)
    for h in range(HEADS):
        s = s + (ops[h][0] + ops[h][1]) * rec[:, h:h + 1]
    colacc[...] += jnp.sum(s, axis=0, keepdims=True)


    @pl.when(i == N // BM - 1)
    def _():
        gm = colacc[...] * (1.0 / (HEADS * N)) + bg_ref[...]
        t = jnp.maximum(
            jnp.dot(gm, wc1_ref[...], preferred_element_type=_f32)
            + bc1_ref[...], 0.0)
        out_ref[...] = (jnp.dot(t, wc2_ref[...], preferred_element_type=_f32)
                        + bc2_ref[...])


def _tc_final(ops, den, bg, wc1, bc1, wc2, bc2):
    return pl.pallas_call(
        _final_body,
        grid=(N // BM,),
        in_specs=[
            *[pl.BlockSpec((2, BM, 128), lambda i: (0, i, 0))
              for _ in range(HEADS)],
            pl.BlockSpec((2, BM, 16), lambda i: (0, i, 0)),
            pl.BlockSpec((1, 128), lambda i: (0, 0)),
            pl.BlockSpec((128, 64), lambda i: (0, 0)),
            pl.BlockSpec((1, 64), lambda i: (0, 0)),
            pl.BlockSpec((64, 2), lambda i: (0, 0)),
            pl.BlockSpec((1, 2), lambda i: (0, 0)),
        ],
        out_specs=pl.BlockSpec((1, 2), lambda i: (0, 0)),
        out_shape=jax.ShapeDtypeStruct((1, 2), _f32),
        scratch_shapes=[pltpu.VMEM((1, 128), _f32)],
    )(*ops, den, bg, wc1, bc1, wc2, bc2)


def kernel(x, edge_index, W1, b1, g1, be1, W2, b2, g2, be2, W3, b3, g3, be3,
           Wg, a_s, a_d, bg, Wc1, bc1, Wc2, bc2):
    loops = jnp.arange(N, dtype=edge_index.dtype)
    pad = EP - (edge_index.shape[1] + N)
    src = jnp.concatenate(
        [edge_index[0], loops, jnp.zeros((pad,), _i32)]).astype(_i32)
    dst = jnp.concatenate(
        [edge_index[1], loops, jnp.full((pad,), DUM, _i32)]).astype(_i32)
    src2d = src.reshape(ERows, 128)
    dst2d = dst.reshape(ERows, 128)

    z16 = jnp.zeros((NP, 16), _f32)
    z128 = jnp.zeros((NP, 128), _f32)
    ones16 = jnp.ones((128, 16), _f32)

    degp = _sc_deg(dst2d, ones16, z16)

    h = x
    for (W, b, g, be) in ((W1, b1, g1, be1), (W2, b2, g2, be2),
                          (W3, b3, g3, be3)):
        hw = _tc_mm_scale(h, W, degp)
        part = _sc_agg(hw, src2d, dst2d, z128)
        z, st = _tc_bnz(part, degp, b.reshape(1, 128))
        h = _tc_apply(z, st, g.reshape(1, 128), be.reshape(1, 128), h)

    eye = jnp.eye(HEADS, dtype=_f32)
    As_s = jnp.reshape(a_s[:, :, None] * eye[:, None, :], (HEADS * C, HEADS))
    As_d = jnp.reshape(a_d[:, :, None] * eye[:, None, :], (HEADS * C, HEADS))
    Ass = jnp.pad(jnp.concatenate([As_s, As_s], axis=1), ((0, 0), (0, 96)))
    Asd = jnp.pad(jnp.concatenate([As_d, As_d], axis=1), ((0, 0), (0, 96)))

    hpad = jnp.pad(h, ((0, NP - N), (0, 0)))
    hh3, esX, edX = _tc_gat_pre(hpad, Wg, Ass, Asd)
    ee = _sc_gat_ee(esX, edX, src2d, dst2d)
    denp = _sc_den(ee, dst2d, z16)
    ops = [_make_gat_head(hd)(hh3[hd], src2d, dst2d, ee, z128)
           for hd in range(HEADS)]
    return _tc_final(ops, denp, bg.reshape(1, 128), Wc1,
                     bc1.reshape(1, 64), Wc2, bc2.reshape(1, 2))


# speedup vs baseline: 2.8714x; 1.0374x over previous
import functools

import jax
import jax.numpy as jnp
from jax import lax
from jax.experimental import pallas as pl
from jax.experimental.pallas import tpu as pltpu
from jax.experimental.pallas import tpu_sc as plsc

N = 10000
D = 128
HID = 128
HEADS = 8
C = 128

NP = 10240
DUM = 10000
EP = 360448
ERows = 2816
JB = ERows // 32
SL = NP // 16
BM = 400
BM2 = 512

_f32 = jnp.float32
_i32 = jnp.int32

_MESH = plsc.VectorSubcoreMesh(core_axis_name="c", subcore_axis_name="s")


def _wid():
    return lax.axis_index("c") * 16 + lax.axis_index("s")


@functools.partial(
    pl.kernel,
    out_type=jax.ShapeDtypeStruct((2, NP, 16), _f32),
    mesh=_MESH,
    scratch_types=[
        pltpu.VMEM((JB, 128), _i32),
        pltpu.VMEM((128, 16), _f32),
        pltpu.VMEM_SHARED((NP, 16), _f32),
    ],
)
def _sc_deg(dst2d, ones16, z16, out, dstv, ones_v, acc):
    c = lax.axis_index("c")
    s = lax.axis_index("s")
    w = _wid()
    pltpu.sync_copy(z16.at[pl.ds(s * SL, SL)], acc.at[pl.ds(s * SL, SL)])
    pltpu.sync_copy(ones16, ones_v)
    pltpu.sync_copy(dst2d.at[pl.ds(w * JB, JB)], dstv)
    plsc.subcore_barrier()

    def body(j, _):
        pltpu.sync_copy(ones_v, acc.at[dstv.at[j]], add=True)
        return ()

    lax.fori_loop(0, JB, body, (), unroll=False)
    plsc.subcore_barrier()
    pltpu.sync_copy(acc.at[pl.ds(s * SL, SL)], out.at[c, pl.ds(s * SL, SL)])


@functools.partial(
    pl.kernel,
    out_type=jax.ShapeDtypeStruct((2, NP, 128), _f32),
    mesh=_MESH,
    scratch_types=[
        pltpu.VMEM((8, 128), _i32),
        pltpu.VMEM((16, 64), _i32),
        pltpu.VMEM((64, 128), _f32),
        pltpu.VMEM((64, 128), _f32),
        pltpu.VMEM_SHARED((NP, 128), _f32),
        pltpu.SemaphoreType.DMA,
        pltpu.SemaphoreType.DMA,
    ],
)
def _sc_agg(table, src2d, dst2d64, z128, out,
            srcv, dstv, ra, rb, acc, sem_a, sem_b):
    c = lax.axis_index("c")
    s = lax.axis_index("s")
    w = _wid()
    pltpu.sync_copy(z128.at[pl.ds(s * SL, SL)], acc.at[pl.ds(s * SL, SL)])
    plsc.subcore_barrier()
    bufs = ((ra, sem_a), (rb, sem_b))

    def group(g, _):
        base = w * JB + g * 8
        pltpu.sync_copy(src2d.at[pl.ds(base, 8)], srcv)
        pltpu.sync_copy(dst2d64.at[pl.ds(2 * base, 16)], dstv)
        descs = [None] * 16
        descs[0] = pltpu.async_copy(
            table.at[srcv.at[0, pl.ds(0, 64)]], ra, sem_a)
        for k in range(16):
            cur, _sem = bufs[k % 2]
            if k < 15:
                nb, nsem = bufs[(k + 1) % 2]
                descs[k + 1] = pltpu.async_copy(
                    table.at[srcv.at[(k + 1) // 2,
                                     pl.ds(((k + 1) % 2) * 64, 64)]],
                    nb, nsem)
            descs[k].wait()
            pltpu.sync_copy(cur, acc.at[dstv.at[k]], add=True)
        return ()

    lax.fori_loop(0, JB // 8, group, (), unroll=False)
    plsc.subcore_barrier()
    pltpu.sync_copy(acc.at[pl.ds(s * SL, SL)], out.at[c, pl.ds(s * SL, SL)])


@functools.partial(
    pl.kernel,
    out_type=jax.ShapeDtypeStruct((EP, 16), _f32),
    mesh=_MESH,
    scratch_types=[
        pltpu.VMEM((8, 128), _i32),
        pltpu.VMEM((8, 128), _i32),
        pltpu.VMEM((128, 128), _f32),
        pltpu.VMEM((128, 128), _f32),
        pltpu.VMEM((128, 128), _f32),
        pltpu.VMEM((128, 128), _f32),
        pltpu.VMEM((128, 16), _f32),
        pltpu.SemaphoreType.DMA,
        pltpu.SemaphoreType.DMA,
        pltpu.SemaphoreType.DMA,
        pltpu.SemaphoreType.DMA,
    ],
)
def _sc_gat_ee(esX, edX, src2d, dst2d, ee_out,
               srcv, dstv, av0, av1, bv0, bv1, eev,
               sa0, sa1, sb0, sb1):
    w = _wid()
    abufs = ((av0, sa0), (av1, sa1))
    bbufs = ((bv0, sb0), (bv1, sb1))

    def group(g, _):
        base = w * JB + g * 8
        pltpu.sync_copy(src2d.at[pl.ds(base, 8)], srcv)
        pltpu.sync_copy(dst2d.at[pl.ds(base, 8)], dstv)
        da = [None] * 8
        db = [None] * 8
        da[0] = pltpu.async_copy(esX.at[srcv.at[0]], av0, sa0)
        db[0] = pltpu.async_copy(edX.at[dstv.at[0]], bv0, sb0)
        for j in range(8):
            av, _sa = abufs[j % 2]
            bv, _sb = bbufs[j % 2]
            if j < 7:
                na, nsa = abufs[(j + 1) % 2]
                nb, nsb = bbufs[(j + 1) % 2]
                da[j + 1] = pltpu.async_copy(esX.at[srcv.at[j + 1]], na, nsa)
                db[j + 1] = pltpu.async_copy(edX.at[dstv.at[j + 1]], nb, nsb)
            da[j].wait()
            db[j].wait()

            def inner(i, _, av=av, bv=bv):
                e = av[i, pl.ds(0, 16)] + bv[i, pl.ds(0, 16)]
                e = jnp.where(e > 0.0, e, e * 0.2)
                eev[i, :] = jnp.exp(e)
                return ()

            lax.fori_loop(0, 128, inner, (), unroll=4)
            pltpu.sync_copy(eev, ee_out.at[pl.ds((base + j) * 128, 128)])
        return ()

    lax.fori_loop(0, JB // 8, group, (), unroll=False)


@functools.partial(
    pl.kernel,
    out_type=jax.ShapeDtypeStruct((2, NP, 16), _f32),
    mesh=_MESH,
    scratch_types=[
        pltpu.VMEM((JB, 128), _i32),
        pltpu.VMEM((128, 16), _f32),
        pltpu.VMEM_SHARED((NP, 16), _f32),
    ],
)
def _sc_den(ee, dst2d, z16, out, dstv, eev, acc):
    c = lax.axis_index("c")
    s = lax.axis_index("s")
    w = _wid()
    pltpu.sync_copy(z16.at[pl.ds(s * SL, SL)], acc.at[pl.ds(s * SL, SL)])
    pltpu.sync_copy(dst2d.at[pl.ds(w * JB, JB)], dstv)
    plsc.subcore_barrier()

    def body(j, _):
        pltpu.sync_copy(ee.at[pl.ds((w * JB + j) * 128, 128)], eev)
        pltpu.sync_copy(eev, acc.at[dstv.at[j]], add=True)
        return ()

    lax.fori_loop(0, JB, body, (), unroll=False)
    plsc.subcore_barrier()
    pltpu.sync_copy(acc.at[pl.ds(s * SL, SL)], out.at[c, pl.ds(s * SL, SL)])


@functools.lru_cache(maxsize=None)
def _make_gat_head(h):
    @functools.partial(
        pl.kernel,
        out_type=jax.ShapeDtypeStruct((2, NP, 128), _f32),
        mesh=_MESH,
        scratch_types=[
            pltpu.VMEM((8, 128), _i32),
            pltpu.VMEM((16, 64), _i32),
            pltpu.VMEM((64, 128), _f32),
            pltpu.VMEM((64, 128), _f32),
            pltpu.VMEM((128, 16), _f32),
            pltpu.VMEM_SHARED((NP, 128), _f32),
            pltpu.SemaphoreType.DMA,
            pltpu.SemaphoreType.DMA,
        ],
        name=f"sc_gat_head{h}",
    )
    def _sc_gat_head(hh_h, src2d, dst2d64, ee, z128, out,
                     srcv, dstv, ra, rb, eev, acc, sem_a, sem_b):
        c = lax.axis_index("c")
        s = lax.axis_index("s")
        w = _wid()
        pltpu.sync_copy(z128.at[pl.ds(s * SL, SL)], acc.at[pl.ds(s * SL, SL)])
        plsc.subcore_barrier()
        bufs = ((ra, sem_a), (rb, sem_b))

        def group(g, _):
            base = w * JB + g * 8
            pltpu.sync_copy(src2d.at[pl.ds(base, 8)], srcv)
            pltpu.sync_copy(dst2d64.at[pl.ds(2 * base, 16)], dstv)
            descs = [None] * 16
            descs[0] = pltpu.async_copy(
                hh_h.at[srcv.at[0, pl.ds(0, 64)]], ra, sem_a)
            for k in range(16):
                cur, _sem = bufs[k % 2]
                if k % 2 == 0:
                    pltpu.sync_copy(
                        ee.at[pl.ds((base + k // 2) * 128, 128)], eev)
                if k < 15:
                    nb, nsem = bufs[(k + 1) % 2]
                    descs[k + 1] = pltpu.async_copy(
                        hh_h.at[srcv.at[(k + 1) // 2,
                                        pl.ds(((k + 1) % 2) * 64, 64)]],
                        nb, nsem)
                descs[k].wait()
                off = (k % 2) * 64

                def inner(i, _, cur=cur, off=off):
                    row = eev[off + i, :]
                    sc = jnp.full((16,), row[h], _f32)
                    for blk in range(8):
                        cur[i, pl.ds(blk * 16, 16)] = (
                            cur[i, pl.ds(blk * 16, 16)] * sc
                        )
                    return ()

                lax.fori_loop(0, 64, inner, (), unroll=2)
                pltpu.sync_copy(cur, acc.at[dstv.at[k]], add=True)
            return ()

        lax.fori_loop(0, JB // 8, group, (), unroll=False)
        plsc.subcore_barrier()
        pltpu.sync_copy(acc.at[pl.ds(s * SL, SL)],
                        out.at[c, pl.ds(s * SL, SL)])

    return _sc_gat_head


def _mm_scale_body(h_ref, w_ref, deg_ref, out_ref):
    deg = deg_ref[0, :, 0:1] + deg_ref[1, :, 0:1]
    dinv = lax.rsqrt(deg)
    out_ref[...] = jnp.dot(h_ref[...], w_ref[...],
                           preferred_element_type=_f32) * dinv


def _tc_mm_scale(h, w, degp):
    return pl.pallas_call(
        _mm_scale_body,
        grid=(N // BM,),
        in_specs=[
            pl.BlockSpec((BM, 128), lambda i: (i, 0)),
            pl.BlockSpec((128, 128), lambda i: (0, 0)),
            pl.BlockSpec((2, BM, 16), lambda i: (0, i, 0)),
        ],
        out_specs=pl.BlockSpec((BM, 128), lambda i: (i, 0)),
        out_shape=jax.ShapeDtypeStruct((N, 128), _f32),
    )(h, w, degp)


def _bnz_body(p_ref, deg_ref, b_ref, z_ref, st_ref):
    i = pl.program_id(0)
    deg = deg_ref[0, :, 0:1] + deg_ref[1, :, 0:1]
    dinv = lax.rsqrt(deg)
    z = dinv * (p_ref[0] + p_ref[1]) + b_ref[...]
    z_ref[...] = z

    @pl.when(i == 0)
    def _():
        st_ref[...] = jnp.zeros((2, 128), _f32)

    s1 = jnp.sum(z, axis=0, keepdims=True)
    s2 = jnp.sum(z * z, axis=0, keepdims=True)
    st_ref[...] += jnp.concatenate([s1, s2], axis=0)


def _tc_bnz(p, degp, b):
    return pl.pallas_call(
        _bnz_body,
        grid=(N // BM,),
        in_specs=[
            pl.BlockSpec((2, BM, 128), lambda i: (0, i, 0)),
            pl.BlockSpec((2, BM, 16), lambda i: (0, i, 0)),
            pl.BlockSpec((1, 128), lambda i: (0, 0)),
        ],
        out_specs=[
            pl.BlockSpec((BM, 128), lambda i: (i, 0)),
            pl.BlockSpec((2, 128), lambda i: (0, 0)),
        ],
        out_shape=[
            jax.ShapeDtypeStruct((N, 128), _f32),
            jax.ShapeDtypeStruct((2, 128), _f32),
        ],
    )(p, degp, b)


def _apply_body(z_ref, st_ref, g_ref, be_ref, hp_ref, out_ref):
    m = st_ref[0:1, :] * (1.0 / N)
    ex2 = st_ref[1:2, :] * (1.0 / N)
    var = ex2 - m * m
    tau = lax.rsqrt(var + 1e-5)
    hn = g_ref[...] * (z_ref[...] - m) * tau + be_ref[...]
    out_ref[...] = hp_ref[...] + jnp.maximum(hn, 0.0)


def _tc_apply(z, st, g, be, hp):
    return pl.pallas_call(
        _apply_body,
        grid=(N // BM,),
        in_specs=[
            pl.BlockSpec((BM, 128), lambda i: (i, 0)),
            pl.BlockSpec((2, 128), lambda i: (0, 0)),
            pl.BlockSpec((1, 128), lambda i: (0, 0)),
            pl.BlockSpec((1, 128), lambda i: (0, 0)),
            pl.BlockSpec((BM, 128), lambda i: (i, 0)),
        ],
        out_specs=pl.BlockSpec((BM, 128), lambda i: (i, 0)),
        out_shape=jax.ShapeDtypeStruct((N, 128), _f32),
    )(z, st, g, be, hp)


def _gat_pre_body(x_ref, wg_ref, ass_ref, asd_ref, hh_ref, es_ref, ed_ref):
    h = pl.program_id(1)
    hh = jnp.dot(x_ref[...], wg_ref[...], preferred_element_type=_f32)
    hh_ref[0] = hh

    @pl.when(h == 0)
    def _():
        es_ref[...] = jnp.zeros((BM2, 128), _f32)
        ed_ref[...] = jnp.zeros((BM2, 128), _f32)

    es_ref[...] += jnp.dot(hh, ass_ref[...], preferred_element_type=_f32)
    ed_ref[...] += jnp.dot(hh, asd_ref[...], preferred_element_type=_f32)


def _tc_gat_pre(x, wg, ass, asd):
    return pl.pallas_call(
        _gat_pre_body,
        grid=(NP // BM2, HEADS),
        in_specs=[
            pl.BlockSpec((BM2, 128), lambda i, h: (i, 0)),
            pl.BlockSpec((128, 128), lambda i, h: (0, h)),
            pl.BlockSpec((128, 128), lambda i, h: (h, 0)),
            pl.BlockSpec((128, 128), lambda i, h: (h, 0)),
        ],
        out_specs=[
            pl.BlockSpec((1, BM2, 128), lambda i, h: (h, i, 0)),
            pl.BlockSpec((BM2, 128), lambda i, h: (i, 0)),
            pl.BlockSpec((BM2, 128), lambda i, h: (i, 0)),
        ],
        out_shape=[
            jax.ShapeDtypeStruct((HEADS, NP, 128), _f32),
            jax.ShapeDtypeStruct((NP, 128), _f32),
            jax.ShapeDtypeStruct((NP, 128), _f32),
        ],
    )(x, wg, ass, asd)


def _final_body(op0, op1, op2, op3, op4, op5, op6, op7,
                den_ref, bg_ref, wc1_ref, bc1_ref, wc2_ref, bc2_ref,
                out_ref, colacc):
    i = pl.program_id(0)
    ops = (op0, op1, op2, op3, op4, op5, op6, op7)

    @pl.when(i == 0)
    def _():
        colacc[...] = jnp.zeros((1, 128), _f32)

    den = den_ref[0, :, 0:8] + den_ref[1, :, 0:8]
    rec = 1.0 / den
    s = jnp.zeros((BM, 128), _f32)
    for h in range(HEADS):
        s = s + (ops[h][0] + ops[h][1]) * rec[:, h:h + 1]
    colacc[...] += jnp.sum(s, axis=0, keepdims=True)


    @pl.when(i == N // BM - 1)
    def _():
        gm = colacc[...] * (1.0 / (HEADS * N)) + bg_ref[...]
        t = jnp.maximum(
            jnp.dot(gm, wc1_ref[...], preferred_element_type=_f32)
            + bc1_ref[...], 0.0)
        out_ref[...] = (jnp.dot(t, wc2_ref[...], preferred_element_type=_f32)
                        + bc2_ref[...])


def _tc_final(ops, den, bg, wc1, bc1, wc2, bc2):
    return pl.pallas_call(
        _final_body,
        grid=(N // BM,),
        in_specs=[
            *[pl.BlockSpec((2, BM, 128), lambda i: (0, i, 0))
              for _ in range(HEADS)],
            pl.BlockSpec((2, BM, 16), lambda i: (0, i, 0)),
            pl.BlockSpec((1, 128), lambda i: (0, 0)),
            pl.BlockSpec((128, 64), lambda i: (0, 0)),
            pl.BlockSpec((1, 64), lambda i: (0, 0)),
            pl.BlockSpec((64, 2), lambda i: (0, 0)),
            pl.BlockSpec((1, 2), lambda i: (0, 0)),
        ],
        out_specs=pl.BlockSpec((1, 2), lambda i: (0, 0)),
        out_shape=jax.ShapeDtypeStruct((1, 2), _f32),
        scratch_shapes=[pltpu.VMEM((1, 128), _f32)],
    )(*ops, den, bg, wc1, bc1, wc2, bc2)


def kernel(x, edge_index, W1, b1, g1, be1, W2, b2, g2, be2, W3, b3, g3, be3,
           Wg, a_s, a_d, bg, Wc1, bc1, Wc2, bc2):
    loops = jnp.arange(N, dtype=edge_index.dtype)
    pad = EP - (edge_index.shape[1] + N)
    src = jnp.concatenate(
        [edge_index[0], loops, jnp.zeros((pad,), _i32)]).astype(_i32)
    dst = jnp.concatenate(
        [edge_index[1], loops, jnp.full((pad,), DUM, _i32)]).astype(_i32)
    src2d = src.reshape(ERows, 128)
    dst2d = dst.reshape(ERows, 128)
    dst2d64 = dst.reshape(EP // 64, 64)

    z16 = jnp.zeros((NP, 16), _f32)
    z128 = jnp.zeros((NP, 128), _f32)
    ones16 = jnp.ones((128, 16), _f32)

    degp = _sc_deg(dst2d, ones16, z16)

    h = x
    for (W, b, g, be) in ((W1, b1, g1, be1), (W2, b2, g2, be2),
                          (W3, b3, g3, be3)):
        hw = _tc_mm_scale(h, W, degp)
        part = _sc_agg(hw, src2d, dst2d64, z128)
        z, st = _tc_bnz(part, degp, b.reshape(1, 128))
        h = _tc_apply(z, st, g.reshape(1, 128), be.reshape(1, 128), h)

    eye = jnp.eye(HEADS, dtype=_f32)
    As_s = jnp.reshape(a_s[:, :, None] * eye[:, None, :], (HEADS * C, HEADS))
    As_d = jnp.reshape(a_d[:, :, None] * eye[:, None, :], (HEADS * C, HEADS))
    Ass = jnp.pad(jnp.concatenate([As_s, As_s], axis=1), ((0, 0), (0, 96)))
    Asd = jnp.pad(jnp.concatenate([As_d, As_d], axis=1), ((0, 0), (0, 96)))

    hpad = jnp.pad(h, ((0, NP - N), (0, 0)))
    hh3, esX, edX = _tc_gat_pre(hpad, Wg, Ass, Asd)
    ee = _sc_gat_ee(esX, edX, src2d, dst2d)
    denp = _sc_den(ee, dst2d, z16)
    ops = [_make_gat_head(hd)(hh3[hd], src2d, dst2d64, ee, z128)
           for hd in range(HEADS)]
    return _tc_final(ops, denp, bg.reshape(1, 128), Wc1,
                     bc1.reshape(1, 64), Wc2, bc2.reshape(1, 2))


# speedup vs baseline: 2.9802x; 1.0379x over previous
import functools

import jax
import jax.numpy as jnp
from jax import lax
from jax.experimental import pallas as pl
from jax.experimental.pallas import tpu as pltpu
from jax.experimental.pallas import tpu_sc as plsc

N = 10000
D = 128
HID = 128
HEADS = 8
C = 128

NP = 10240
DUM = 10000
EP = 360448
ERows = 2816
JB = ERows // 32
SL = NP // 16
BM = 400
BM2 = 512

_f32 = jnp.float32
_i32 = jnp.int32

_MESH = plsc.VectorSubcoreMesh(core_axis_name="c", subcore_axis_name="s")


def _wid():
    return lax.axis_index("c") * 16 + lax.axis_index("s")


@functools.partial(
    pl.kernel,
    out_type=jax.ShapeDtypeStruct((2, NP, 16), _f32),
    mesh=_MESH,
    scratch_types=[
        pltpu.VMEM((JB, 128), _i32),
        pltpu.VMEM((128, 16), _f32),
        pltpu.VMEM_SHARED((NP, 16), _f32),
    ],
)
def _sc_deg(dst2d, ones16, z16, out, dstv, ones_v, acc):
    c = lax.axis_index("c")
    s = lax.axis_index("s")
    w = _wid()
    pltpu.sync_copy(z16.at[pl.ds(s * SL, SL)], acc.at[pl.ds(s * SL, SL)])
    pltpu.sync_copy(ones16, ones_v)
    pltpu.sync_copy(dst2d.at[pl.ds(w * JB, JB)], dstv)
    plsc.subcore_barrier()

    def body(j, _):
        pltpu.sync_copy(ones_v, acc.at[dstv.at[j]], add=True)
        return ()

    lax.fori_loop(0, JB, body, (), unroll=False)
    plsc.subcore_barrier()
    pltpu.sync_copy(acc.at[pl.ds(s * SL, SL)], out.at[c, pl.ds(s * SL, SL)])


@functools.partial(
    pl.kernel,
    out_type=jax.ShapeDtypeStruct((2, NP, 128), _f32),
    mesh=_MESH,
    scratch_types=[
        pltpu.VMEM((8, 128), _i32),
        pltpu.VMEM((8, 128), _i32),
        pltpu.VMEM((128, 128), _f32),
        pltpu.VMEM_SHARED((NP, 128), _f32),
        pltpu.SemaphoreType.DMA,
    ],
)
def _sc_agg(table, src2d, dst2d, z128, out,
            srcv, dstv, rows, acc, sem):
    c = lax.axis_index("c")
    s = lax.axis_index("s")
    w = _wid()
    pltpu.sync_copy(z128.at[pl.ds(s * SL, SL)], acc.at[pl.ds(s * SL, SL)])
    plsc.subcore_barrier()

    def group(g, _):
        base = w * JB + g * 8
        pltpu.sync_copy(src2d.at[pl.ds(base, 8)], srcv)
        pltpu.sync_copy(dst2d.at[pl.ds(base, 8)], dstv)

        def body(j, _):
            pltpu.async_copy(table.at[srcv.at[j]], rows, sem).wait()
            pltpu.sync_copy(rows, acc.at[dstv.at[j]], add=True)
            return ()

        lax.fori_loop(0, 8, body, (), unroll=False)
        return ()

    lax.fori_loop(0, JB // 8, group, (), unroll=False)
    plsc.subcore_barrier()
    pltpu.sync_copy(acc.at[pl.ds(s * SL, SL)], out.at[c, pl.ds(s * SL, SL)])


@functools.partial(
    pl.kernel,
    out_type=jax.ShapeDtypeStruct((EP, 16), _f32),
    mesh=_MESH,
    scratch_types=[
        pltpu.VMEM((8, 128), _i32),
        pltpu.VMEM((8, 128), _i32),
        pltpu.VMEM((128, 128), _f32),
        pltpu.VMEM((128, 128), _f32),
        pltpu.VMEM((128, 128), _f32),
        pltpu.VMEM((128, 128), _f32),
        pltpu.VMEM((128, 16), _f32),
        pltpu.SemaphoreType.DMA,
        pltpu.SemaphoreType.DMA,
        pltpu.SemaphoreType.DMA,
        pltpu.SemaphoreType.DMA,
    ],
)
def _sc_gat_ee(esX, edX, src2d, dst2d, ee_out,
               srcv, dstv, av0, av1, bv0, bv1, eev,
               sa0, sa1, sb0, sb1):
    w = _wid()
    abufs = ((av0, sa0), (av1, sa1))
    bbufs = ((bv0, sb0), (bv1, sb1))

    def group(g, _):
        base = w * JB + g * 8
        pltpu.sync_copy(src2d.at[pl.ds(base, 8)], srcv)
        pltpu.sync_copy(dst2d.at[pl.ds(base, 8)], dstv)
        da = [None] * 8
        db = [None] * 8
        da[0] = pltpu.async_copy(esX.at[srcv.at[0]], av0, sa0)
        db[0] = pltpu.async_copy(edX.at[dstv.at[0]], bv0, sb0)
        for j in range(8):
            av, _sa = abufs[j % 2]
            bv, _sb = bbufs[j % 2]
            if j < 7:
                na, nsa = abufs[(j + 1) % 2]
                nb, nsb = bbufs[(j + 1) % 2]
                da[j + 1] = pltpu.async_copy(esX.at[srcv.at[j + 1]], na, nsa)
                db[j + 1] = pltpu.async_copy(edX.at[dstv.at[j + 1]], nb, nsb)
            da[j].wait()
            db[j].wait()

            def inner(i, _, av=av, bv=bv):
                e = av[i, pl.ds(0, 16)] + bv[i, pl.ds(0, 16)]
                e = jnp.where(e > 0.0, e, e * 0.2)
                eev[i, :] = jnp.exp(e)
                return ()

            lax.fori_loop(0, 128, inner, (), unroll=4)
            pltpu.sync_copy(eev, ee_out.at[pl.ds((base + j) * 128, 128)])
        return ()

    lax.fori_loop(0, JB // 8, group, (), unroll=False)


@functools.partial(
    pl.kernel,
    out_type=jax.ShapeDtypeStruct((2, NP, 16), _f32),
    mesh=_MESH,
    scratch_types=[
        pltpu.VMEM((JB, 128), _i32),
        pltpu.VMEM((128, 16), _f32),
        pltpu.VMEM_SHARED((NP, 16), _f32),
    ],
)
def _sc_den(ee, dst2d, z16, out, dstv, eev, acc):
    c = lax.axis_index("c")
    s = lax.axis_index("s")
    w = _wid()
    pltpu.sync_copy(z16.at[pl.ds(s * SL, SL)], acc.at[pl.ds(s * SL, SL)])
    pltpu.sync_copy(dst2d.at[pl.ds(w * JB, JB)], dstv)
    plsc.subcore_barrier()

    def body(j, _):
        pltpu.sync_copy(ee.at[pl.ds((w * JB + j) * 128, 128)], eev)
        pltpu.sync_copy(eev, acc.at[dstv.at[j]], add=True)
        return ()

    lax.fori_loop(0, JB, body, (), unroll=False)
    plsc.subcore_barrier()
    pltpu.sync_copy(acc.at[pl.ds(s * SL, SL)], out.at[c, pl.ds(s * SL, SL)])


@functools.lru_cache(maxsize=None)
def _make_gat_head(h):
    @functools.partial(
        pl.kernel,
        out_type=jax.ShapeDtypeStruct((2, NP, 128), _f32),
        mesh=_MESH,
        scratch_types=[
            pltpu.VMEM((8, 128), _i32),
            pltpu.VMEM((16, 64), _i32),
            pltpu.VMEM((64, 128), _f32),
            pltpu.VMEM((64, 128), _f32),
            pltpu.VMEM((128, 16), _f32),
            pltpu.VMEM_SHARED((NP, 128), _f32),
            pltpu.SemaphoreType.DMA,
            pltpu.SemaphoreType.DMA,
        ],
        name=f"sc_gat_head{h}",
    )
    def _sc_gat_head(hh_h, src2d, dst2d64, ee, z128, out,
                     srcv, dstv, ra, rb, eev, acc, sem_a, sem_b):
        c = lax.axis_index("c")
        s = lax.axis_index("s")
        w = _wid()
        pltpu.sync_copy(z128.at[pl.ds(s * SL, SL)], acc.at[pl.ds(s * SL, SL)])
        plsc.subcore_barrier()
        bufs = ((ra, sem_a), (rb, sem_b))

        def group(g, _):
            base = w * JB + g * 8
            pltpu.sync_copy(src2d.at[pl.ds(base, 8)], srcv)
            pltpu.sync_copy(dst2d64.at[pl.ds(2 * base, 16)], dstv)
            descs = [None] * 16
            descs[0] = pltpu.async_copy(
                hh_h.at[srcv.at[0, pl.ds(0, 64)]], ra, sem_a)
            for k in range(16):
                cur, _sem = bufs[k % 2]
                if k % 2 == 0:
                    pltpu.sync_copy(
                        ee.at[pl.ds((base + k // 2) * 128, 128)], eev)
                if k < 15:
                    nb, nsem = bufs[(k + 1) % 2]
                    descs[k + 1] = pltpu.async_copy(
                        hh_h.at[srcv.at[(k + 1) // 2,
                                        pl.ds(((k + 1) % 2) * 64, 64)]],
                        nb, nsem)
                descs[k].wait()
                off = (k % 2) * 64

                def inner(i, _, cur=cur, off=off):
                    row = eev[off + i, :]
                    sc = jnp.full((16,), row[h], _f32)
                    for blk in range(8):
                        cur[i, pl.ds(blk * 16, 16)] = (
                            cur[i, pl.ds(blk * 16, 16)] * sc
                        )
                    return ()

                lax.fori_loop(0, 64, inner, (), unroll=2)
                pltpu.sync_copy(cur, acc.at[dstv.at[k]], add=True)
            return ()

        lax.fori_loop(0, JB // 8, group, (), unroll=False)
        plsc.subcore_barrier()
        pltpu.sync_copy(acc.at[pl.ds(s * SL, SL)],
                        out.at[c, pl.ds(s * SL, SL)])

    return _sc_gat_head


def _mm_scale_body(h_ref, w_ref, deg_ref, out_ref):
    deg = deg_ref[0, :, 0:1] + deg_ref[1, :, 0:1]
    dinv = lax.rsqrt(deg)
    out_ref[...] = jnp.dot(h_ref[...], w_ref[...],
                           preferred_element_type=_f32) * dinv


def _tc_mm_scale(h, w, degp):
    return pl.pallas_call(
        _mm_scale_body,
        grid=(N // BM,),
        in_specs=[
            pl.BlockSpec((BM, 128), lambda i: (i, 0)),
            pl.BlockSpec((128, 128), lambda i: (0, 0)),
            pl.BlockSpec((2, BM, 16), lambda i: (0, i, 0)),
        ],
        out_specs=pl.BlockSpec((BM, 128), lambda i: (i, 0)),
        out_shape=jax.ShapeDtypeStruct((N, 128), _f32),
    )(h, w, degp)


def _bnz_body(p_ref, deg_ref, b_ref, z_ref, st_ref):
    i = pl.program_id(0)
    deg = deg_ref[0, :, 0:1] + deg_ref[1, :, 0:1]
    dinv = lax.rsqrt(deg)
    z = dinv * (p_ref[0] + p_ref[1]) + b_ref[...]
    z_ref[...] = z

    @pl.when(i == 0)
    def _():
        st_ref[...] = jnp.zeros((2, 128), _f32)

    s1 = jnp.sum(z, axis=0, keepdims=True)
    s2 = jnp.sum(z * z, axis=0, keepdims=True)
    st_ref[...] += jnp.concatenate([s1, s2], axis=0)


def _tc_bnz(p, degp, b):
    return pl.pallas_call(
        _bnz_body,
        grid=(N // BM,),
        in_specs=[
            pl.BlockSpec((2, BM, 128), lambda i: (0, i, 0)),
            pl.BlockSpec((2, BM, 16), lambda i: (0, i, 0)),
            pl.BlockSpec((1, 128), lambda i: (0, 0)),
        ],
        out_specs=[
            pl.BlockSpec((BM, 128), lambda i: (i, 0)),
            pl.BlockSpec((2, 128), lambda i: (0, 0)),
        ],
        out_shape=[
            jax.ShapeDtypeStruct((N, 128), _f32),
            jax.ShapeDtypeStruct((2, 128), _f32),
        ],
    )(p, degp, b)


def _apply_body(z_ref, st_ref, g_ref, be_ref, hp_ref, out_ref):
    m = st_ref[0:1, :] * (1.0 / N)
    ex2 = st_ref[1:2, :] * (1.0 / N)
    var = ex2 - m * m
    tau = lax.rsqrt(var + 1e-5)
    hn = g_ref[...] * (z_ref[...] - m) * tau + be_ref[...]
    out_ref[...] = hp_ref[...] + jnp.maximum(hn, 0.0)


def _tc_apply(z, st, g, be, hp):
    return pl.pallas_call(
        _apply_body,
        grid=(N // BM,),
        in_specs=[
            pl.BlockSpec((BM, 128), lambda i: (i, 0)),
            pl.BlockSpec((2, 128), lambda i: (0, 0)),
            pl.BlockSpec((1, 128), lambda i: (0, 0)),
            pl.BlockSpec((1, 128), lambda i: (0, 0)),
            pl.BlockSpec((BM, 128), lambda i: (i, 0)),
        ],
        out_specs=pl.BlockSpec((BM, 128), lambda i: (i, 0)),
        out_shape=jax.ShapeDtypeStruct((N, 128), _f32),
    )(z, st, g, be, hp)


def _gat_pre_body(x_ref, wg_ref, ass_ref, asd_ref, hh_ref, es_ref, ed_ref):
    h = pl.program_id(1)
    hh = jnp.dot(x_ref[...], wg_ref[...], preferred_element_type=_f32)
    hh_ref[0] = hh

    @pl.when(h == 0)
    def _():
        es_ref[...] = jnp.zeros((BM2, 128), _f32)
        ed_ref[...] = jnp.zeros((BM2, 128), _f32)

    es_ref[...] += jnp.dot(hh, ass_ref[...], preferred_element_type=_f32)
    ed_ref[...] += jnp.dot(hh, asd_ref[...], preferred_element_type=_f32)


def _tc_gat_pre(x, wg, ass, asd):
    return pl.pallas_call(
        _gat_pre_body,
        grid=(NP // BM2, HEADS),
        in_specs=[
            pl.BlockSpec((BM2, 128), lambda i, h: (i, 0)),
            pl.BlockSpec((128, 128), lambda i, h: (0, h)),
            pl.BlockSpec((128, 128), lambda i, h: (h, 0)),
            pl.BlockSpec((128, 128), lambda i, h: (h, 0)),
        ],
        out_specs=[
            pl.BlockSpec((1, BM2, 128), lambda i, h: (h, i, 0)),
            pl.BlockSpec((BM2, 128), lambda i, h: (i, 0)),
            pl.BlockSpec((BM2, 128), lambda i, h: (i, 0)),
        ],
        out_shape=[
            jax.ShapeDtypeStruct((HEADS, NP, 128), _f32),
            jax.ShapeDtypeStruct((NP, 128), _f32),
            jax.ShapeDtypeStruct((NP, 128), _f32),
        ],
    )(x, wg, ass, asd)


def _final_body(op0, op1, op2, op3, op4, op5, op6, op7,
                den_ref, bg_ref, wc1_ref, bc1_ref, wc2_ref, bc2_ref,
                out_ref, colacc):
    i = pl.program_id(0)
    ops = (op0, op1, op2, op3, op4, op5, op6, op7)

    @pl.when(i == 0)
    def _():
        colacc[...] = jnp.zeros((1, 128), _f32)

    den = den_ref[0, :, 0:8] + den_ref[1, :, 0:8]
    rec = 1.0 / den
    s = jnp.zeros((BM, 128), _f32)
    for h in range(HEADS):
        s = s + (ops[h][0] + ops[h][1]) * rec[:, h:h + 1]
    colacc[...] += jnp.sum(s, axis=0, keepdims=True)


    @pl.when(i == N // BM - 1)
    def _():
        gm = colacc[...] * (1.0 / (HEADS * N)) + bg_ref[...]
        t = jnp.maximum(
            jnp.dot(gm, wc1_ref[...], preferred_element_type=_f32)
            + bc1_ref[...], 0.0)
        out_ref[...] = (jnp.dot(t, wc2_ref[...], preferred_element_type=_f32)
                        + bc2_ref[...])


def _tc_final(ops, den, bg, wc1, bc1, wc2, bc2):
    return pl.pallas_call(
        _final_body,
        grid=(N // BM,),
        in_specs=[
            *[pl.BlockSpec((2, BM, 128), lambda i: (0, i, 0))
              for _ in range(HEADS)],
            pl.BlockSpec((2, BM, 16), lambda i: (0, i, 0)),
            pl.BlockSpec((1, 128), lambda i: (0, 0)),
            pl.BlockSpec((128, 64), lambda i: (0, 0)),
            pl.BlockSpec((1, 64), lambda i: (0, 0)),
            pl.BlockSpec((64, 2), lambda i: (0, 0)),
            pl.BlockSpec((1, 2), lambda i: (0, 0)),
        ],
        out_specs=pl.BlockSpec((1, 2), lambda i: (0, 0)),
        out_shape=jax.ShapeDtypeStruct((1, 2), _f32),
        scratch_shapes=[pltpu.VMEM((1, 128), _f32)],
    )(*ops, den, bg, wc1, bc1, wc2, bc2)


def kernel(x, edge_index, W1, b1, g1, be1, W2, b2, g2, be2, W3, b3, g3, be3,
           Wg, a_s, a_d, bg, Wc1, bc1, Wc2, bc2):
    loops = jnp.arange(N, dtype=edge_index.dtype)
    pad = EP - (edge_index.shape[1] + N)
    src = jnp.concatenate(
        [edge_index[0], loops, jnp.zeros((pad,), _i32)]).astype(_i32)
    dst = jnp.concatenate(
        [edge_index[1], loops, jnp.full((pad,), DUM, _i32)]).astype(_i32)
    src2d = src.reshape(ERows, 128)
    dst2d = dst.reshape(ERows, 128)
    dst2d64 = dst.reshape(EP // 64, 64)

    z16 = jnp.zeros((NP, 16), _f32)
    z128 = jnp.zeros((NP, 128), _f32)
    ones16 = jnp.ones((128, 16), _f32)

    degp = _sc_deg(dst2d, ones16, z16)

    h = x
    for (W, b, g, be) in ((W1, b1, g1, be1), (W2, b2, g2, be2),
                          (W3, b3, g3, be3)):
        hw = _tc_mm_scale(h, W, degp)
        part = _sc_agg(hw, src2d, dst2d, z128)
        z, st = _tc_bnz(part, degp, b.reshape(1, 128))
        h = _tc_apply(z, st, g.reshape(1, 128), be.reshape(1, 128), h)

    eye = jnp.eye(HEADS, dtype=_f32)
    As_s = jnp.reshape(a_s[:, :, None] * eye[:, None, :], (HEADS * C, HEADS))
    As_d = jnp.reshape(a_d[:, :, None] * eye[:, None, :], (HEADS * C, HEADS))
    Ass = jnp.pad(jnp.concatenate([As_s, As_s], axis=1), ((0, 0), (0, 96)))
    Asd = jnp.pad(jnp.concatenate([As_d, As_d], axis=1), ((0, 0), (0, 96)))

    hpad = jnp.pad(h, ((0, NP - N), (0, 0)))
    hh3, esX, edX = _tc_gat_pre(hpad, Wg, Ass, Asd)
    ee = _sc_gat_ee(esX, edX, src2d, dst2d)
    denp = _sc_den(ee, dst2d, z16)
    ops = [_make_gat_head(hd)(hh3[hd], src2d, dst2d64, ee, z128)
           for hd in range(HEADS)]
    return _tc_final(ops, denp, bg.reshape(1, 128), Wc1,
                     bc1.reshape(1, 64), Wc2, bc2.reshape(1, 2))
